# baseline (speedup 1.0000x reference)
import jax
import jax.numpy as jnp
from jax import lax
from jax.experimental import pallas as pl
from jax.experimental.pallas import tpu as pltpu
from jax.experimental.pallas import tpu_sc as plsc

N = 10000
NP = 10240
NPH = NP // 2
NE = 160000
NEP = 163840
F = 128
D = 256
NMOL = 2000
NMOLP = 2048
BATCH = 4096
PADN = NP - 1
PADG = NMOLP - 1

NC = 2
NS = 16
L = 16

_mesh = plsc.VectorSubcoreMesh(core_axis_name="c", subcore_axis_name="s")
_params = pltpu.CompilerParams(needs_layout_passes=False)
f32 = jnp.float32
i32 = jnp.int32


def _pats():
    io = lax.iota(i32, L)
    pat2 = lax.shift_right_logical(io, 3)
    pat8 = lax.bitwise_and(io, 7)
    return pat2, pat8


def _pats4():
    io = lax.iota(i32, L)
    pdiv = lax.shift_right_logical(io, 2)
    pmod = lax.bitwise_and(io, 3)
    return pdiv, pmod


def _zero_flat(acc, n):
    z = jnp.zeros((L,), f32)

    def body(i, _):
        acc[pl.ds(i * L, L)] = z
        return 0

    lax.fori_loop(0, n // L, body, 0)


def _deg_body(edges_hbm, out_hbm, edgebuf, cnt):
    c = lax.axis_index("c")
    s = lax.axis_index("s")

    def z(i, _):
        cnt[pl.ds(i * L, L)] = jnp.zeros((L,), f32)
        return 0

    lax.fori_loop(0, NP // L, z, 0)
    pltpu.sync_copy(edges_hbm.at[pl.ds(c * 1280 + s * 80, 80)], edgebuf)
    ones = jnp.full((L,), 1.0, f32)

    def body(r, _):
        for q in range(8):
            v = edgebuf[r, pl.ds(q * L, L)]
            plsc.addupdate_scatter(cnt, [v], ones)
        return 0

    lax.fori_loop(0, 80, body, 0)
    pltpu.sync_copy(cnt, out_hbm.at[pl.ds((c * NS + s) * NP, NP)])


@jax.jit
def _sc_degrees(edges2d):
    return pl.kernel(
        _deg_body,
        out_type=jax.ShapeDtypeStruct((32 * NP,), f32),
        mesh=_mesh,
        scratch_types=[
            pltpu.VMEM((80, 128), i32),
            pltpu.VMEM((NP,), f32),
        ],
        compiler_params=_params,
    )(edges2d)


def _hist_body(nf_ref, deg_ref, h_ref, norms_ref):
    degp = deg_ref[...]
    d_src = jnp.sum(degp[:NS], axis=0)
    d_dst = jnp.sum(degp[NS:], axis=0)
    ns = jnp.where(d_src > 0, lax.rsqrt(jnp.maximum(d_src, 1.0)), 0.0)
    nd = jnp.where(d_dst > 0, lax.rsqrt(jnp.maximum(d_dst, 1.0)), 0.0)
    norms_ref[...] = jnp.stack([ns, nd], axis=0)
    feat = nf_ref[...]
    io = lax.broadcasted_iota(i32, (feat.shape[0], F), 1)
    acc = jnp.zeros((feat.shape[0], F), f32)
    for j in range(10):
        acc = acc + (feat[:, j : j + 1] == io).astype(f32)
    h_ref[...] = acc * ns[:, None]


@jax.jit
def _tc_hist(nf_p, deg):
    blk = 1024
    return pl.pallas_call(
        _hist_body,
        grid=(NP // blk,),
        in_specs=[
            pl.BlockSpec((blk, 10), lambda j: (j, 0)),
            pl.BlockSpec((32, blk), lambda j: (0, j)),
        ],
        out_specs=[
            pl.BlockSpec((blk, F), lambda j: (j, 0)),
            pl.BlockSpec((2, blk), lambda j: (0, j)),
        ],
        out_shape=[
            jax.ShapeDtypeStruct((NP, F), f32),
            jax.ShapeDtypeStruct((2, NP), f32),
        ],
    )(nf_p, deg)


def _accum_edges(src2d, dst2d, hsl, acc, srcb, dstb):
    pdiv, pmod = _pats4()

    def blk(b, _):
        pltpu.sync_copy(src2d.at[pl.ds(b * 40, 40)], srcb)
        pltpu.sync_copy(dst2d.at[pl.ds(b * 40, 40)], dstb)

        def row(g, _):
            rowv = jnp.zeros((L,), i32) + g
            for e in range(32):
                colv = pdiv + 4 * e
                se = plsc.load_gather(srcb, [rowv, colv])
                de = plsc.load_gather(dstb, [rowv, colv])
                val = plsc.load_gather(hsl, [se * 4 + pmod])
                plsc.addupdate_scatter(acc, [de * 4 + pmod], val)
            return 0

        lax.fori_loop(0, 40, row, 0)
        return 0

    lax.fori_loop(0, 32, blk, 0)


SLW = 4 * NP


def _agg1_body(hp, src2d, dst2d, out_hbm, hsl, acc, srcb, dstb):
    c = lax.axis_index("c")
    s = lax.axis_index("s")
    q = c * NS + s
    pltpu.sync_copy(hp.at[pl.ds(q * SLW, SLW)], hsl)
    _zero_flat(acc, SLW)
    _accum_edges(src2d, dst2d, hsl, acc, srcb, dstb)
    pltpu.sync_copy(acc, out_hbm.at[pl.ds(q * SLW, SLW)])


@jax.jit
def _sc_agg1(hp, src2d, dst2d):
    return pl.kernel(
        _agg1_body,
        out_type=jax.ShapeDtypeStruct((32 * SLW,), f32),
        mesh=_mesh,
        scratch_types=[
            pltpu.VMEM((SLW,), f32),
            pltpu.VMEM((SLW,), f32),
            pltpu.VMEM((40, 128), i32),
            pltpu.VMEM((40, 128), i32),
        ],
        compiler_params=_params,
    )(hp, src2d, dst2d)


def _agg2_body(hp, src2d, dst2d, out_hbm, hsl, acc, srcb, dstb):
    c = lax.axis_index("c")
    s = lax.axis_index("s")
    q = c * NS + s
    for r in range(2):
        q2 = q + 32 * r
        pltpu.sync_copy(hp.at[pl.ds(q2 * SLW, SLW)], hsl)
        _zero_flat(acc, SLW)
        _accum_edges(src2d, dst2d, hsl, acc, srcb, dstb)
        pltpu.sync_copy(acc, out_hbm.at[pl.ds(q2 * SLW, SLW)])


@jax.jit
def _sc_agg2(hp, src2d, dst2d):
    return pl.kernel(
        _agg2_body,
        out_type=jax.ShapeDtypeStruct((64 * SLW,), f32),
        mesh=_mesh,
        scratch_types=[
            pltpu.VMEM((SLW,), f32),
            pltpu.VMEM((SLW,), f32),
            pltpu.VMEM((40, 128), i32),
            pltpu.VMEM((40, 128), i32),
        ],
        compiler_params=_params,
    )(hp, src2d, dst2d)


def _l1_body(p_ref, norms_ref, w_ref, b_ref, out_ref):
    nm = norms_ref[...]
    agg = p_ref[...] * nm[1][:, None]
    x = jnp.dot(agg, w_ref[...], preferred_element_type=f32) + b_ref[...]
    out_ref[...] = jnp.maximum(x, 0.0) * nm[0][:, None]


@jax.jit
def _tc_l1(P, norms, W1, b1):
    blk = 256
    return pl.pallas_call(
        _l1_body,
        grid=(NP // blk,),
        in_specs=[
            pl.BlockSpec((blk, 128), lambda j: (j, 0)),
            pl.BlockSpec((2, blk), lambda j: (0, j)),
            pl.BlockSpec((F, D), lambda j: (0, 0)),
            pl.BlockSpec((1, D), lambda j: (0, 0)),
        ],
        out_specs=pl.BlockSpec((blk, D), lambda j: (j, 0)),
        out_shape=jax.ShapeDtypeStruct((NP, D), f32),
    )(P, norms, W1, b1)


def _l2_body(a_ref, norms_ref, w_ref, b_ref, h2_ref, s_ref):
    j = pl.program_id(0)
    nm = norms_ref[...]
    nd = nm[1][:, None]
    h2 = jnp.dot(a_ref[...] * nd, w_ref[...], preferred_element_type=f32) \
        + b_ref[...]
    h2_ref[...] = h2
    rn = jnp.sqrt(jnp.sum(h2 * h2, axis=1))
    rows = j * h2.shape[0] + lax.broadcasted_iota(i32, rn.shape, 0)
    contrib = jnp.sum(jnp.where(rows < N, rn, 0.0))

    @pl.when(j == 0)
    def _():
        s_ref[...] = jnp.zeros((1, 1), f32)

    s_ref[...] += jnp.full((1, 1), 1.0, f32) * contrib


@jax.jit
def _tc_l2(A2, norms, W2, b2):
    blk = 256
    return pl.pallas_call(
        _l2_body,
        grid=(NP // blk,),
        in_specs=[
            pl.BlockSpec((blk, D), lambda j: (j, 0)),
            pl.BlockSpec((2, blk), lambda j: (0, j)),
            pl.BlockSpec((D, D), lambda j: (0, 0)),
            pl.BlockSpec((1, D), lambda j: (0, 0)),
        ],
        out_specs=[
            pl.BlockSpec((blk, D), lambda j: (j, 0)),
            pl.BlockSpec((1, 1), lambda j: (0, 0)),
        ],
        out_shape=[
            jax.ShapeDtypeStruct((NP, D), f32),
            jax.ShapeDtypeStruct((1, 1), f32),
        ],
    )(A2, norms, W2, b2)


def _segsum_body(h2_hbm, gid_hbm, out_hbm, sbuf, gidb, acc):
    c = lax.axis_index("c")
    s = lax.axis_index("s")
    q = c * NS + s
    _zero_flat(acc, 8 * NMOLP)
    pltpu.sync_copy(h2_hbm.at[pl.ds(q * 8 * NP, 8 * NP)], sbuf)
    pltpu.sync_copy(gid_hbm, gidb)
    pat2, pat8 = _pats()

    def body(jj, _):
        rv = pat2 + 2 * jj
        gv = plsc.load_gather(gidb, [rv])
        val = sbuf[pl.ds(jj * L, L)]
        plsc.addupdate_scatter(acc, [gv * 8 + pat8], val)
        return 0

    lax.fori_loop(0, NP // 2, body, 0)
    pltpu.sync_copy(acc, out_hbm.at[pl.ds(q * 8 * NMOLP, 8 * NMOLP)])


@jax.jit
def _sc_segsum(h2t, gid1d):
    return pl.kernel(
        _segsum_body,
        out_type=jax.ShapeDtypeStruct((32 * 8 * NMOLP,), f32),
        mesh=_mesh,
        scratch_types=[
            pltpu.VMEM((8 * NP,), f32),
            pltpu.VMEM((NP,), i32),
            pltpu.VMEM((8 * NMOLP,), f32),
        ],
        compiler_params=_params,
    )(h2t, gid1d)


def _combine_body(p_ref, s_ref, out_ref):
    factor = (16.0 * N) / s_ref[0, 0]
    out_ref[...] = p_ref[...] * factor


@jax.jit
def _tc_combine(embP, S):
    blk = 256
    return pl.pallas_call(
        _combine_body,
        grid=(NMOLP // blk,),
        in_specs=[
            pl.BlockSpec((blk, D), lambda j: (j, 0)),
            pl.BlockSpec((1, 1), lambda j: (0, 0)),
        ],
        out_specs=pl.BlockSpec((blk, D), lambda j: (j, 0)),
        out_shape=jax.ShapeDtypeStruct((NMOLP, D), f32),
    )(embP, S)


def _moltable_body(emb_hbm, adj_hbm, out_hbm, adjb, selfb, neib, outb, sem):
    c = lax.axis_index("c")
    s = lax.axis_index("s")
    w = c * NS + s
    pltpu.sync_copy(adj_hbm.at[pl.ds(w * 512, 512)], adjb)
    pltpu.sync_copy(emb_hbm.at[pl.ds(w * 64, 64)], selfb)
    for q in range(4):
        pltpu.async_copy(
            emb_hbm.at[adjb.at[pl.ds(q * 128, 128)]], neib, sem
        ).wait()

        def item(b16, _):
            b = q * 16 + b16
            for v in range(16):
                sl = pl.ds(v * L, L)
                acc = neib[b16 * 8, sl]
                for jj in range(1, 8):
                    acc = acc + neib[b16 * 8 + jj, sl]
                outb[b, sl] = selfb[b, sl] * 2.0 + acc * 0.125
            return 0

        lax.fori_loop(0, 16, item, 0)
    pltpu.sync_copy(outb, out_hbm.at[pl.ds(w * 64, 64)])


@jax.jit
def _sc_moltable(emb, flatadj):
    return pl.kernel(
        _moltable_body,
        out_type=jax.ShapeDtypeStruct((NMOLP, D), f32),
        mesh=_mesh,
        scratch_types=[
            pltpu.VMEM((512,), i32),
            pltpu.VMEM((64, D), f32),
            pltpu.VMEM((128, D), f32),
            pltpu.VMEM((64, D), f32),
            pltpu.SemaphoreType.DMA,
        ],
        compiler_params=_params,
    )(emb, flatadj)


def _bgather_body(g_hbm, ids_hbm, out_hbm, idsb, gbuf, sem):
    c = lax.axis_index("c")
    s = lax.axis_index("s")
    w = c * NS + s
    pltpu.sync_copy(ids_hbm.at[pl.ds(w * 256, 256)], idsb)
    for q in range(2):
        pltpu.async_copy(
            g_hbm.at[idsb.at[pl.ds(q * 128, 128)]], gbuf, sem
        ).wait()
        pltpu.sync_copy(gbuf, out_hbm.at[pl.ds(w * 256 + q * 128, 128)])


@jax.jit
def _sc_bgather(G, ids):
    return pl.kernel(
        _bgather_body,
        out_type=jax.ShapeDtypeStruct((2 * BATCH, D), f32),
        mesh=_mesh,
        scratch_types=[
            pltpu.VMEM((256,), i32),
            pltpu.VMEM((128, D), f32),
            pltpu.SemaphoreType.DMA,
        ],
        compiler_params=_params,
    )(G, ids)


def _pack4(x, ncols):
    return (
        x.reshape(NPH, 2, ncols // 4, 4)
        .transpose(2, 0, 1, 3)
        .reshape(ncols // 4, NPH, 8)
    )


def _unpack4(xp, ncols):
    return (
        xp.reshape(ncols // 4, NPH, 2, 4)
        .transpose(1, 2, 0, 3)
        .reshape(NP, ncols)
    )


def kernel(r_id, p_id, r_bond, p_bond, node_feature, edge_index, graph_id,
           adj_graph, bond_feature, W1, b1, W2, b2, rel_para):
    src = edge_index[0].astype(i32)
    dst = edge_index[1].astype(i32)
    pad = jnp.full((NEP - NE,), PADN, i32)
    src2d = jnp.concatenate([src, pad]).reshape(NEP // 128, 128)
    dst2d = jnp.concatenate([dst, pad]).reshape(NEP // 128, 128)
    edges2d = jnp.concatenate([src2d, dst2d], axis=0)
    nf_p = jnp.pad(node_feature.astype(i32), ((0, NP - N), (0, 0)))
    gid1d = jnp.concatenate(
        [graph_id.astype(i32), jnp.full((NP - N,), PADG, i32)]
    )
    flatadj = jnp.pad(
        adj_graph.astype(i32), ((0, NMOLP - NMOL), (0, 0))
    ).reshape(-1)
    ids = jnp.concatenate([r_id, p_id]).astype(i32)

    deg = _sc_degrees(edges2d)
    h, norms = _tc_hist(nf_p, deg.reshape(32, NP))
    P = _sc_agg1(_pack4(h, F).reshape(-1), src2d, dst2d)
    h1 = _tc_l1(_unpack4(P.reshape(32, NPH, 8), F), norms, W1,
                b1.reshape(1, D))
    A2 = _sc_agg2(_pack4(h1, D).reshape(-1), src2d, dst2d)
    h2, S = _tc_l2(_unpack4(A2.reshape(64, NPH, 8), D), norms, W2,
                   b2.reshape(1, D))
    h2t = h2.reshape(NP, 32, 8).transpose(1, 0, 2).reshape(-1)
    embP = _sc_segsum(h2t, gid1d)
    emb = embP.reshape(32, NMOLP, 8).transpose(1, 0, 2).reshape(NMOLP, D)
    emb = _tc_combine(emb, S)
    G = _sc_moltable(emb, flatadj)
    out = _sc_bgather(G, ids)
    return out[:BATCH], out[BATCH:]


# speedup vs baseline: 2.6740x; 2.6740x over previous
import jax
import jax.numpy as jnp
from jax import lax
from jax.experimental import pallas as pl
from jax.experimental.pallas import tpu as pltpu
from jax.experimental.pallas import tpu_sc as plsc

N = 10000
NP = 10240
NPH = NP // 2
NE = 160000
NEP = 163840
F = 128
D = 256
NMOL = 2000
NMOLP = 2048
BATCH = 4096
PADN = NP - 1
PADG = NMOLP - 1

NC = 2
NS = 16
L = 16

_mesh = plsc.VectorSubcoreMesh(core_axis_name="c", subcore_axis_name="s")
_params = pltpu.CompilerParams(needs_layout_passes=False)
f32 = jnp.float32
i32 = jnp.int32


def _pats():
    io = lax.iota(i32, L)
    pat2 = lax.shift_right_logical(io, 3)
    pat8 = lax.bitwise_and(io, 7)
    return pat2, pat8


def _pats4():
    io = lax.iota(i32, L)
    pdiv = lax.shift_right_logical(io, 2)
    pmod = lax.bitwise_and(io, 3)
    return pdiv, pmod


def _zero_flat(acc, n):
    z = jnp.zeros((L,), f32)

    def body(i, _):
        acc[pl.ds(i * L, L)] = z
        return 0

    lax.fori_loop(0, n // L, body, 0)


def _deg_body(edges_hbm, out_hbm, edgebuf, cnt):
    c = lax.axis_index("c")
    s = lax.axis_index("s")

    def z(i, _):
        cnt[pl.ds(i * L, L)] = jnp.zeros((L,), f32)
        return 0

    lax.fori_loop(0, NP // L, z, 0)
    pltpu.sync_copy(edges_hbm.at[pl.ds(c * 1280 + s * 80, 80)], edgebuf)
    ones = jnp.full((L,), 1.0, f32)

    def body(r, _):
        for q in range(8):
            v = edgebuf[r, pl.ds(q * L, L)]
            plsc.addupdate_scatter(cnt, [v], ones)
        return 0

    lax.fori_loop(0, 80, body, 0)
    pltpu.sync_copy(cnt, out_hbm.at[pl.ds((c * NS + s) * NP, NP)])


@jax.jit
def _sc_degrees(edges2d):
    return pl.kernel(
        _deg_body,
        out_type=jax.ShapeDtypeStruct((32 * NP,), f32),
        mesh=_mesh,
        scratch_types=[
            pltpu.VMEM((80, 128), i32),
            pltpu.VMEM((NP,), f32),
        ],
        compiler_params=_params,
    )(edges2d)


def _hist_body(nft_ref, deg_ref, h_ref, norms_ref):
    degp = deg_ref[...]
    d_src = jnp.sum(degp[:NS], axis=0)
    d_dst = jnp.sum(degp[NS:], axis=0)
    ns = jnp.where(d_src > 0, lax.rsqrt(jnp.maximum(d_src, 1.0)), 0.0)
    nd = jnp.where(d_dst > 0, lax.rsqrt(jnp.maximum(d_dst, 1.0)), 0.0)
    norms_ref[...] = jnp.stack([ns, nd], axis=0)
    feat = nft_ref[...]
    io = lax.broadcasted_iota(i32, (F, feat.shape[1]), 0)
    acc = jnp.zeros((F, feat.shape[1]), f32)
    for j in range(10):
        acc = acc + (feat[j : j + 1, :] == io).astype(f32)
    h_ref[...] = acc * ns[None, :]


@jax.jit
def _tc_hist(nft, deg):
    blk = 2048
    return pl.pallas_call(
        _hist_body,
        grid=(NP // blk,),
        in_specs=[
            pl.BlockSpec((10, blk), lambda j: (0, j)),
            pl.BlockSpec((32, blk), lambda j: (0, j)),
        ],
        out_specs=[
            pl.BlockSpec((F, blk), lambda j: (0, j)),
            pl.BlockSpec((2, blk), lambda j: (0, j)),
        ],
        out_shape=[
            jax.ShapeDtypeStruct((F, NP), f32),
            jax.ShapeDtypeStruct((2, NP), f32),
        ],
    )(nft, deg)


def _accum_edges(src2d, dst2d, hsl, acc, srcb, dstb):
    pdiv, pmod = _pats4()
    pnp = pmod * NP

    def blk(b, _):
        pltpu.sync_copy(src2d.at[pl.ds(b * 40, 40)], srcb)
        pltpu.sync_copy(dst2d.at[pl.ds(b * 40, 40)], dstb)

        def row(g, _):
            rowv = jnp.zeros((L,), i32) + g
            for e in range(32):
                colv = pdiv + 4 * e
                se = plsc.load_gather(srcb, [rowv, colv])
                de = plsc.load_gather(dstb, [rowv, colv])
                val = plsc.load_gather(hsl, [se + pnp])
                plsc.addupdate_scatter(acc, [de + pnp], val)
            return 0

        lax.fori_loop(0, 40, row, 0)
        return 0

    lax.fori_loop(0, 32, blk, 0)


SLW = 4 * NP


def _agg1_body(hp, src2d, dst2d, out_hbm, hsl, acc, srcb, dstb):
    c = lax.axis_index("c")
    s = lax.axis_index("s")
    q = c * NS + s
    pltpu.sync_copy(hp.at[pl.ds(q * SLW, SLW)], hsl)
    _zero_flat(acc, SLW)
    _accum_edges(src2d, dst2d, hsl, acc, srcb, dstb)
    pltpu.sync_copy(acc, out_hbm.at[pl.ds(q * SLW, SLW)])


@jax.jit
def _sc_agg1(hp, src2d, dst2d):
    return pl.kernel(
        _agg1_body,
        out_type=jax.ShapeDtypeStruct((32 * SLW,), f32),
        mesh=_mesh,
        scratch_types=[
            pltpu.VMEM((SLW,), f32),
            pltpu.VMEM((SLW,), f32),
            pltpu.VMEM((40, 128), i32),
            pltpu.VMEM((40, 128), i32),
        ],
        compiler_params=_params,
    )(hp, src2d, dst2d)


def _agg2_body(hp, src2d, dst2d, out_hbm, hsl, acc, srcb, dstb):
    c = lax.axis_index("c")
    s = lax.axis_index("s")
    q = c * NS + s
    for r in range(2):
        q2 = q + 32 * r
        pltpu.sync_copy(hp.at[pl.ds(q2 * SLW, SLW)], hsl)
        _zero_flat(acc, SLW)
        _accum_edges(src2d, dst2d, hsl, acc, srcb, dstb)
        pltpu.sync_copy(acc, out_hbm.at[pl.ds(q2 * SLW, SLW)])


@jax.jit
def _sc_agg2(hp, src2d, dst2d):
    return pl.kernel(
        _agg2_body,
        out_type=jax.ShapeDtypeStruct((64 * SLW,), f32),
        mesh=_mesh,
        scratch_types=[
            pltpu.VMEM((SLW,), f32),
            pltpu.VMEM((SLW,), f32),
            pltpu.VMEM((40, 128), i32),
            pltpu.VMEM((40, 128), i32),
        ],
        compiler_params=_params,
    )(hp, src2d, dst2d)


def _l1_body(p_ref, norms_ref, w_ref, b_ref, out_ref):
    nm = norms_ref[...]
    agg = p_ref[...] * nm[1][None, :]
    x = lax.dot_general(
        w_ref[...], agg, (((0,), (0,)), ((), ())),
        preferred_element_type=f32,
    ) + b_ref[...].reshape(D, 1)
    out_ref[...] = jnp.maximum(x, 0.0) * nm[0][None, :]


@jax.jit
def _tc_l1(P, norms, W1, b1):
    blk = 1024
    return pl.pallas_call(
        _l1_body,
        grid=(NP // blk,),
        in_specs=[
            pl.BlockSpec((F, blk), lambda j: (0, j)),
            pl.BlockSpec((2, blk), lambda j: (0, j)),
            pl.BlockSpec((F, D), lambda j: (0, 0)),
            pl.BlockSpec((1, D), lambda j: (0, 0)),
        ],
        out_specs=pl.BlockSpec((D, blk), lambda j: (0, j)),
        out_shape=jax.ShapeDtypeStruct((D, NP), f32),
    )(P, norms, W1, b1)


def _l2_body(a_ref, norms_ref, w_ref, b_ref, h2_ref, s_ref):
    j = pl.program_id(0)
    nm = norms_ref[...]
    h2 = lax.dot_general(
        w_ref[...], a_ref[...] * nm[1][None, :], (((0,), (0,)), ((), ())),
        preferred_element_type=f32,
    ) + b_ref[...].reshape(D, 1)
    h2_ref[...] = h2
    rn = jnp.sqrt(jnp.sum(h2 * h2, axis=0))
    cols = j * h2.shape[1] + lax.broadcasted_iota(i32, rn.shape, 0)
    contrib = jnp.sum(jnp.where(cols < N, rn, 0.0))

    @pl.when(j == 0)
    def _():
        s_ref[...] = jnp.zeros((1, 1), f32)

    s_ref[...] += jnp.full((1, 1), 1.0, f32) * contrib


@jax.jit
def _tc_l2(A2, norms, W2, b2):
    blk = 1024
    return pl.pallas_call(
        _l2_body,
        grid=(NP // blk,),
        in_specs=[
            pl.BlockSpec((D, blk), lambda j: (0, j)),
            pl.BlockSpec((2, blk), lambda j: (0, j)),
            pl.BlockSpec((D, D), lambda j: (0, 0)),
            pl.BlockSpec((1, D), lambda j: (0, 0)),
        ],
        out_specs=[
            pl.BlockSpec((D, blk), lambda j: (0, j)),
            pl.BlockSpec((1, 1), lambda j: (0, 0)),
        ],
        out_shape=[
            jax.ShapeDtypeStruct((D, NP), f32),
            jax.ShapeDtypeStruct((1, 1), f32),
        ],
    )(A2, norms, W2, b2)


def _segsum_body(h2_hbm, gid_hbm, out_hbm, sbuf, gidb, acc):
    c = lax.axis_index("c")
    s = lax.axis_index("s")
    q = c * NS + s
    _zero_flat(acc, 8 * NMOLP)
    pltpu.sync_copy(h2_hbm.at[pl.ds(q * 8 * NP, 8 * NP)], sbuf)
    pltpu.sync_copy(gid_hbm, gidb)
    pat2, pat8 = _pats()
    base = pat8 * NP + pat2

    def body(jj, _):
        rv = pat2 + 2 * jj
        gv = plsc.load_gather(gidb, [rv])
        val = plsc.load_gather(sbuf, [base + 2 * jj])
        plsc.addupdate_scatter(acc, [gv * 8 + pat8], val)
        return 0

    lax.fori_loop(0, NP // 2, body, 0)
    pltpu.sync_copy(acc, out_hbm.at[pl.ds(q * 8 * NMOLP, 8 * NMOLP)])


@jax.jit
def _sc_segsum(h2t, gid1d):
    return pl.kernel(
        _segsum_body,
        out_type=jax.ShapeDtypeStruct((32 * 8 * NMOLP,), f32),
        mesh=_mesh,
        scratch_types=[
            pltpu.VMEM((8 * NP,), f32),
            pltpu.VMEM((NP,), i32),
            pltpu.VMEM((8 * NMOLP,), f32),
        ],
        compiler_params=_params,
    )(h2t, gid1d)


def _combine_body(p_ref, s_ref, out_ref):
    factor = (16.0 * N) / s_ref[0, 0]
    out_ref[...] = p_ref[...] * factor


@jax.jit
def _tc_combine(embP, S):
    blk = 256
    return pl.pallas_call(
        _combine_body,
        grid=(NMOLP // blk,),
        in_specs=[
            pl.BlockSpec((blk, D), lambda j: (j, 0)),
            pl.BlockSpec((1, 1), lambda j: (0, 0)),
        ],
        out_specs=pl.BlockSpec((blk, D), lambda j: (j, 0)),
        out_shape=jax.ShapeDtypeStruct((NMOLP, D), f32),
    )(embP, S)


def _moltable_body(emb_hbm, adj_hbm, out_hbm, adjb, selfb, neib, outb, sem):
    c = lax.axis_index("c")
    s = lax.axis_index("s")
    w = c * NS + s
    pltpu.sync_copy(adj_hbm.at[pl.ds(w * 512, 512)], adjb)
    pltpu.sync_copy(emb_hbm.at[pl.ds(w * 64, 64)], selfb)
    for q in range(4):
        pltpu.async_copy(
            emb_hbm.at[adjb.at[pl.ds(q * 128, 128)]], neib, sem
        ).wait()

        def item(b16, _):
            b = q * 16 + b16
            for v in range(16):
                sl = pl.ds(v * L, L)
                acc = neib[b16 * 8, sl]
                for jj in range(1, 8):
                    acc = acc + neib[b16 * 8 + jj, sl]
                outb[b, sl] = selfb[b, sl] * 2.0 + acc * 0.125
            return 0

        lax.fori_loop(0, 16, item, 0)
    pltpu.sync_copy(outb, out_hbm.at[pl.ds(w * 64, 64)])


@jax.jit
def _sc_moltable(emb, flatadj):
    return pl.kernel(
        _moltable_body,
        out_type=jax.ShapeDtypeStruct((NMOLP, D), f32),
        mesh=_mesh,
        scratch_types=[
            pltpu.VMEM((512,), i32),
            pltpu.VMEM((64, D), f32),
            pltpu.VMEM((128, D), f32),
            pltpu.VMEM((64, D), f32),
            pltpu.SemaphoreType.DMA,
        ],
        compiler_params=_params,
    )(emb, flatadj)


def _bgather_body(g_hbm, ids_hbm, out_hbm, idsb, gbuf, sem):
    c = lax.axis_index("c")
    s = lax.axis_index("s")
    w = c * NS + s
    pltpu.sync_copy(ids_hbm.at[pl.ds(w * 256, 256)], idsb)
    for q in range(2):
        pltpu.async_copy(
            g_hbm.at[idsb.at[pl.ds(q * 128, 128)]], gbuf, sem
        ).wait()
        pltpu.sync_copy(gbuf, out_hbm.at[pl.ds(w * 256 + q * 128, 128)])


@jax.jit
def _sc_bgather(G, ids):
    return pl.kernel(
        _bgather_body,
        out_type=jax.ShapeDtypeStruct((2 * BATCH, D), f32),
        mesh=_mesh,
        scratch_types=[
            pltpu.VMEM((256,), i32),
            pltpu.VMEM((128, D), f32),
            pltpu.SemaphoreType.DMA,
        ],
        compiler_params=_params,
    )(G, ids)


def kernel(r_id, p_id, r_bond, p_bond, node_feature, edge_index, graph_id,
           adj_graph, bond_feature, W1, b1, W2, b2, rel_para):
    src = edge_index[0].astype(i32)
    dst = edge_index[1].astype(i32)
    pad = jnp.full((NEP - NE,), PADN, i32)
    src2d = jnp.concatenate([src, pad]).reshape(NEP // 128, 128)
    dst2d = jnp.concatenate([dst, pad]).reshape(NEP // 128, 128)
    edges2d = jnp.concatenate([src2d, dst2d], axis=0)
    nf_p = jnp.pad(node_feature.astype(i32), ((0, NP - N), (0, 0)))
    gid1d = jnp.concatenate(
        [graph_id.astype(i32), jnp.full((NP - N,), PADG, i32)]
    )
    flatadj = jnp.pad(
        adj_graph.astype(i32), ((0, NMOLP - NMOL), (0, 0))
    ).reshape(-1)
    ids = jnp.concatenate([r_id, p_id]).astype(i32)

    deg = _sc_degrees(edges2d)
    hT, norms = _tc_hist(nf_p.T, deg.reshape(32, NP))
    P = _sc_agg1(hT.reshape(-1), src2d, dst2d)
    h1T = _tc_l1(P.reshape(F, NP), norms, W1, b1.reshape(1, D))
    A2 = _sc_agg2(h1T.reshape(-1), src2d, dst2d)
    h2T, S = _tc_l2(A2.reshape(D, NP), norms, W2, b2.reshape(1, D))
    embP = _sc_segsum(h2T.reshape(-1), gid1d)
    emb = embP.reshape(32, NMOLP, 8).transpose(1, 0, 2).reshape(NMOLP, D)
    emb = _tc_combine(emb, S)
    G = _sc_moltable(emb, flatadj)
    out = _sc_bgather(G, ids)
    return out[:BATCH], out[BATCH:]


# speedup vs baseline: 3.3624x; 1.2574x over previous
import jax
import jax.numpy as jnp
from jax import lax
from jax.experimental import pallas as pl
from jax.experimental.pallas import tpu as pltpu
from jax.experimental.pallas import tpu_sc as plsc

N = 10000
NP = 10240
NPH = NP // 2
NE = 160000
NEP = 163840
F = 128
D = 256
NMOL = 2000
NMOLP = 2048
BATCH = 4096
PADN = NP - 1
PADG = NMOLP - 1

NC = 2
NS = 16
L = 16

_mesh = plsc.VectorSubcoreMesh(core_axis_name="c", subcore_axis_name="s")
_params = pltpu.CompilerParams(needs_layout_passes=False)
f32 = jnp.float32
i32 = jnp.int32


def _pats():
    io = lax.iota(i32, L)
    pat2 = lax.shift_right_logical(io, 3)
    pat8 = lax.bitwise_and(io, 7)
    return pat2, pat8


def _pats4():
    io = lax.iota(i32, L)
    pdiv = lax.shift_right_logical(io, 2)
    pmod = lax.bitwise_and(io, 3)
    return pdiv, pmod


def _zero_flat(acc, n):
    z = jnp.zeros((L,), f32)

    def body(i, _):
        acc[pl.ds(i * L, L)] = z
        return 0

    lax.fori_loop(0, n // L, body, 0)


def _deg_body(edges_hbm, out_hbm, edgebuf, cnt):
    c = lax.axis_index("c")
    s = lax.axis_index("s")

    def z(i, _):
        cnt[pl.ds(i * L, L)] = jnp.zeros((L,), f32)
        return 0

    lax.fori_loop(0, NP // L, z, 0)
    pltpu.sync_copy(edges_hbm.at[pl.ds(c * 1280 + s * 80, 80)], edgebuf)
    ones = jnp.full((L,), 1.0, f32)

    def body(r, _):
        for q in range(8):
            v = edgebuf[r, pl.ds(q * L, L)]
            plsc.addupdate_scatter(cnt, [v], ones)
        return 0

    lax.fori_loop(0, 80, body, 0)
    pltpu.sync_copy(cnt, out_hbm.at[pl.ds((c * NS + s) * NP, NP)])


@jax.jit
def _sc_degrees(edges2d):
    return pl.kernel(
        _deg_body,
        out_type=jax.ShapeDtypeStruct((32 * NP,), f32),
        mesh=_mesh,
        scratch_types=[
            pltpu.VMEM((80, 128), i32),
            pltpu.VMEM((NP,), f32),
        ],
        compiler_params=_params,
    )(edges2d)


def _hist_body(nft_ref, deg_ref, h_ref, norms_ref):
    degp = deg_ref[...]
    d_src = jnp.sum(degp[:NS], axis=0)
    d_dst = jnp.sum(degp[NS:], axis=0)
    ns = jnp.where(d_src > 0, lax.rsqrt(jnp.maximum(d_src, 1.0)), 0.0)
    nd = jnp.where(d_dst > 0, lax.rsqrt(jnp.maximum(d_dst, 1.0)), 0.0)
    norms_ref[...] = jnp.stack([ns, nd], axis=0)
    feat = nft_ref[...]
    io = lax.broadcasted_iota(i32, (F, feat.shape[1]), 0)
    acc = jnp.zeros((F, feat.shape[1]), f32)
    for j in range(10):
        acc = acc + (feat[j : j + 1, :] == io).astype(f32)
    h_ref[...] = acc * ns[None, :]


@jax.jit
def _tc_hist(nft, deg):
    blk = 2048
    return pl.pallas_call(
        _hist_body,
        grid=(NP // blk,),
        in_specs=[
            pl.BlockSpec((10, blk), lambda j: (0, j)),
            pl.BlockSpec((32, blk), lambda j: (0, j)),
        ],
        out_specs=[
            pl.BlockSpec((F, blk), lambda j: (0, j)),
            pl.BlockSpec((2, blk), lambda j: (0, j)),
        ],
        out_shape=[
            jax.ShapeDtypeStruct((F, NP), f32),
            jax.ShapeDtypeStruct((2, NP), f32),
        ],
    )(nft, deg)


def _accum_edges(src2d, dst2d, hsl, acc, srcb, dstb):
    pdiv, pmod = _pats4()
    pnp = pmod * SAG

    def blk(b, _):
        pltpu.sync_copy(src2d.at[pl.ds(b * 40, 40)], srcb)
        pltpu.sync_copy(dst2d.at[pl.ds(b * 40, 40)], dstb)

        def row(g, _):
            rowv = jnp.zeros((L,), i32) + g
            for e in range(32):
                colv = pdiv + 4 * e
                se = plsc.load_gather(srcb, [rowv, colv])
                de = plsc.load_gather(dstb, [rowv, colv])
                val = plsc.load_gather(hsl, [se + pnp])
                plsc.addupdate_scatter(acc, [de + pnp], val)
            return 0

        lax.fori_loop(0, 40, row, 0)
        return 0

    lax.fori_loop(0, 32, blk, 0)


SLW = 4 * NP
SAG = NP + 8
SBUFW = 4 * SAG


def _ldsl(hbm, buf, q):
    for k in range(4):
        pltpu.sync_copy(hbm.at[pl.ds((q * 4 + k) * NP, NP)],
                        buf.at[pl.ds(k * SAG, NP)])


def _stsl(buf, hbm, q):
    for k in range(4):
        pltpu.sync_copy(buf.at[pl.ds(k * SAG, NP)],
                        hbm.at[pl.ds((q * 4 + k) * NP, NP)])


def _agg1_body(hp, src2d, dst2d, out_hbm, hsl, acc, srcb, dstb):
    c = lax.axis_index("c")
    s = lax.axis_index("s")
    q = c * NS + s
    _ldsl(hp, hsl, q)
    _zero_flat(acc, SBUFW)
    _accum_edges(src2d, dst2d, hsl, acc, srcb, dstb)
    _stsl(acc, out_hbm, q)


@jax.jit
def _sc_agg1(hp, src2d, dst2d):
    return pl.kernel(
        _agg1_body,
        out_type=jax.ShapeDtypeStruct((32 * SLW,), f32),
        mesh=_mesh,
        scratch_types=[
            pltpu.VMEM((SBUFW,), f32),
            pltpu.VMEM((SBUFW,), f32),
            pltpu.VMEM((40, 128), i32),
            pltpu.VMEM((40, 128), i32),
        ],
        compiler_params=_params,
    )(hp, src2d, dst2d)


def _agg2_body(hp, src2d, dst2d, out_hbm, hsl, acc, srcb, dstb):
    c = lax.axis_index("c")
    s = lax.axis_index("s")
    q = c * NS + s
    for r in range(2):
        q2 = q + 32 * r
        _ldsl(hp, hsl, q2)
        _zero_flat(acc, SBUFW)
        _accum_edges(src2d, dst2d, hsl, acc, srcb, dstb)
        _stsl(acc, out_hbm, q2)


@jax.jit
def _sc_agg2(hp, src2d, dst2d):
    return pl.kernel(
        _agg2_body,
        out_type=jax.ShapeDtypeStruct((64 * SLW,), f32),
        mesh=_mesh,
        scratch_types=[
            pltpu.VMEM((SBUFW,), f32),
            pltpu.VMEM((SBUFW,), f32),
            pltpu.VMEM((40, 128), i32),
            pltpu.VMEM((40, 128), i32),
        ],
        compiler_params=_params,
    )(hp, src2d, dst2d)


def _l1_body(p_ref, norms_ref, w_ref, b_ref, out_ref):
    nm = norms_ref[...]
    agg = p_ref[...] * nm[1][None, :]
    x = lax.dot_general(
        w_ref[...], agg, (((0,), (0,)), ((), ())),
        preferred_element_type=f32,
    ) + b_ref[...].reshape(D, 1)
    out_ref[...] = jnp.maximum(x, 0.0) * nm[0][None, :]


@jax.jit
def _tc_l1(P, norms, W1, b1):
    blk = 1024
    return pl.pallas_call(
        _l1_body,
        grid=(NP // blk,),
        in_specs=[
            pl.BlockSpec((F, blk), lambda j: (0, j)),
            pl.BlockSpec((2, blk), lambda j: (0, j)),
            pl.BlockSpec((F, D), lambda j: (0, 0)),
            pl.BlockSpec((1, D), lambda j: (0, 0)),
        ],
        out_specs=pl.BlockSpec((D, blk), lambda j: (0, j)),
        out_shape=jax.ShapeDtypeStruct((D, NP), f32),
    )(P, norms, W1, b1)


def _l2_body(a_ref, norms_ref, w_ref, b_ref, h2_ref, s_ref):
    j = pl.program_id(0)
    nm = norms_ref[...]
    h2 = lax.dot_general(
        w_ref[...], a_ref[...] * nm[1][None, :], (((0,), (0,)), ((), ())),
        preferred_element_type=f32,
    ) + b_ref[...].reshape(D, 1)
    h2_ref[...] = h2
    rn = jnp.sqrt(jnp.sum(h2 * h2, axis=0))
    cols = j * h2.shape[1] + lax.broadcasted_iota(i32, rn.shape, 0)
    contrib = jnp.sum(jnp.where(cols < N, rn, 0.0))

    @pl.when(j == 0)
    def _():
        s_ref[...] = jnp.zeros((1, 1), f32)

    s_ref[...] += jnp.full((1, 1), 1.0, f32) * contrib


@jax.jit
def _tc_l2(A2, norms, W2, b2):
    blk = 1024
    return pl.pallas_call(
        _l2_body,
        grid=(NP // blk,),
        in_specs=[
            pl.BlockSpec((D, blk), lambda j: (0, j)),
            pl.BlockSpec((2, blk), lambda j: (0, j)),
            pl.BlockSpec((D, D), lambda j: (0, 0)),
            pl.BlockSpec((1, D), lambda j: (0, 0)),
        ],
        out_specs=[
            pl.BlockSpec((D, blk), lambda j: (0, j)),
            pl.BlockSpec((1, 1), lambda j: (0, 0)),
        ],
        out_shape=[
            jax.ShapeDtypeStruct((D, NP), f32),
            jax.ShapeDtypeStruct((1, 1), f32),
        ],
    )(A2, norms, W2, b2)


def _segsum_body(h2_hbm, gid_hbm, out_hbm, sbuf, gidb, acc):
    c = lax.axis_index("c")
    s = lax.axis_index("s")
    q = c * NS + s
    _zero_flat(acc, 8 * NMOLP)
    for k in range(8):
        pltpu.sync_copy(h2_hbm.at[pl.ds((q * 8 + k) * NP, NP)],
                        sbuf.at[pl.ds(k * SAG, NP)])
    pltpu.sync_copy(gid_hbm, gidb)
    pat2, pat8 = _pats()
    base = pat8 * SAG + pat2

    def body(jj, _):
        rv = pat2 + 2 * jj
        gv = plsc.load_gather(gidb, [rv])
        val = plsc.load_gather(sbuf, [base + 2 * jj])
        plsc.addupdate_scatter(acc, [gv * 8 + pat8], val)
        return 0

    lax.fori_loop(0, NP // 2, body, 0)
    pltpu.sync_copy(acc, out_hbm.at[pl.ds(q * 8 * NMOLP, 8 * NMOLP)])


@jax.jit
def _sc_segsum(h2t, gid1d):
    return pl.kernel(
        _segsum_body,
        out_type=jax.ShapeDtypeStruct((32 * 8 * NMOLP,), f32),
        mesh=_mesh,
        scratch_types=[
            pltpu.VMEM((8 * SAG,), f32),
            pltpu.VMEM((NP,), i32),
            pltpu.VMEM((8 * NMOLP,), f32),
        ],
        compiler_params=_params,
    )(h2t, gid1d)


def _combine_body(p_ref, s_ref, out_ref):
    factor = (16.0 * N) / s_ref[0, 0]
    out_ref[...] = p_ref[...] * factor


@jax.jit
def _tc_combine(embP, S):
    blk = 256
    return pl.pallas_call(
        _combine_body,
        grid=(NMOLP // blk,),
        in_specs=[
            pl.BlockSpec((blk, D), lambda j: (j, 0)),
            pl.BlockSpec((1, 1), lambda j: (0, 0)),
        ],
        out_specs=pl.BlockSpec((blk, D), lambda j: (j, 0)),
        out_shape=jax.ShapeDtypeStruct((NMOLP, D), f32),
    )(embP, S)


def _moltable_body(emb_hbm, adj_hbm, out_hbm, adjb, selfb, neib, outb, sem):
    c = lax.axis_index("c")
    s = lax.axis_index("s")
    w = c * NS + s
    pltpu.sync_copy(adj_hbm.at[pl.ds(w * 512, 512)], adjb)
    pltpu.sync_copy(emb_hbm.at[pl.ds(w * 64, 64)], selfb)
    for q in range(4):
        pltpu.async_copy(
            emb_hbm.at[adjb.at[pl.ds(q * 128, 128)]], neib, sem
        ).wait()

        def item(b16, _):
            b = q * 16 + b16
            for v in range(16):
                sl = pl.ds(v * L, L)
                acc = neib[b16 * 8, sl]
                for jj in range(1, 8):
                    acc = acc + neib[b16 * 8 + jj, sl]
                outb[b, sl] = selfb[b, sl] * 2.0 + acc * 0.125
            return 0

        lax.fori_loop(0, 16, item, 0)
    pltpu.sync_copy(outb, out_hbm.at[pl.ds(w * 64, 64)])


@jax.jit
def _sc_moltable(emb, flatadj):
    return pl.kernel(
        _moltable_body,
        out_type=jax.ShapeDtypeStruct((NMOLP, D), f32),
        mesh=_mesh,
        scratch_types=[
            pltpu.VMEM((512,), i32),
            pltpu.VMEM((64, D), f32),
            pltpu.VMEM((128, D), f32),
            pltpu.VMEM((64, D), f32),
            pltpu.SemaphoreType.DMA,
        ],
        compiler_params=_params,
    )(emb, flatadj)


def _bgather_body(g_hbm, ids_hbm, out_hbm, idsb, gbuf, sem):
    c = lax.axis_index("c")
    s = lax.axis_index("s")
    w = c * NS + s
    pltpu.sync_copy(ids_hbm.at[pl.ds(w * 256, 256)], idsb)
    for q in range(2):
        pltpu.async_copy(
            g_hbm.at[idsb.at[pl.ds(q * 128, 128)]], gbuf, sem
        ).wait()
        pltpu.sync_copy(gbuf, out_hbm.at[pl.ds(w * 256 + q * 128, 128)])


@jax.jit
def _sc_bgather(G, ids):
    return pl.kernel(
        _bgather_body,
        out_type=jax.ShapeDtypeStruct((2 * BATCH, D), f32),
        mesh=_mesh,
        scratch_types=[
            pltpu.VMEM((256,), i32),
            pltpu.VMEM((128, D), f32),
            pltpu.SemaphoreType.DMA,
        ],
        compiler_params=_params,
    )(G, ids)


def kernel(r_id, p_id, r_bond, p_bond, node_feature, edge_index, graph_id,
           adj_graph, bond_feature, W1, b1, W2, b2, rel_para):
    src = edge_index[0].astype(i32)
    dst = edge_index[1].astype(i32)
    pad = jnp.full((NEP - NE,), PADN, i32)
    src2d = jnp.concatenate([src, pad]).reshape(NEP // 128, 128)
    dst2d = jnp.concatenate([dst, pad]).reshape(NEP // 128, 128)
    edges2d = jnp.concatenate([src2d, dst2d], axis=0)
    nf_p = jnp.pad(node_feature.astype(i32), ((0, NP - N), (0, 0)))
    gid1d = jnp.concatenate(
        [graph_id.astype(i32), jnp.full((NP - N,), PADG, i32)]
    )
    flatadj = jnp.pad(
        adj_graph.astype(i32), ((0, NMOLP - NMOL), (0, 0))
    ).reshape(-1)
    ids = jnp.concatenate([r_id, p_id]).astype(i32)

    deg = _sc_degrees(edges2d)
    hT, norms = _tc_hist(nf_p.T, deg.reshape(32, NP))
    P = _sc_agg1(hT.reshape(-1), src2d, dst2d)
    h1T = _tc_l1(P.reshape(F, NP), norms, W1, b1.reshape(1, D))
    A2 = _sc_agg2(h1T.reshape(-1), src2d, dst2d)
    h2T, S = _tc_l2(A2.reshape(D, NP), norms, W2, b2.reshape(1, D))
    embP = _sc_segsum(h2T.reshape(-1), gid1d)
    emb = embP.reshape(32, NMOLP, 8).transpose(1, 0, 2).reshape(NMOLP, D)
    emb = _tc_combine(emb, S)
    G = _sc_moltable(emb, flatadj)
    out = _sc_bgather(G, ids)
    return out[:BATCH], out[BATCH:]


# speedup vs baseline: 3.6724x; 1.0922x over previous
import jax
import jax.numpy as jnp
from jax import lax
from jax.experimental import pallas as pl
from jax.experimental.pallas import tpu as pltpu
from jax.experimental.pallas import tpu_sc as plsc

N = 10000
NP = 10240
NPH = NP // 2
NE = 160000
NEP = 163840
F = 128
D = 256
NMOL = 2000
NMOLP = 2048
BATCH = 4096
PADN = NP - 1
PADG = NMOLP - 1

NC = 2
NS = 16
L = 16

_mesh = plsc.VectorSubcoreMesh(core_axis_name="c", subcore_axis_name="s")
_params = pltpu.CompilerParams(needs_layout_passes=False)
f32 = jnp.float32
i32 = jnp.int32


def _pats():
    io = lax.iota(i32, L)
    pat2 = lax.shift_right_logical(io, 3)
    pat8 = lax.bitwise_and(io, 7)
    return pat2, pat8


def _pats4():
    io = lax.iota(i32, L)
    pdiv = lax.shift_right_logical(io, 2)
    pmod = lax.bitwise_and(io, 3)
    return pdiv, pmod


def _zero_flat(acc, n):
    z = jnp.zeros((L,), f32)

    def body(i, _):
        acc[pl.ds(i * L, L)] = z
        return 0

    lax.fori_loop(0, n // L, body, 0)


def _deg_body(edges_hbm, out_hbm, edgebuf, cnt):
    c = lax.axis_index("c")
    s = lax.axis_index("s")

    def z(i, _):
        cnt[pl.ds(i * L, L)] = jnp.zeros((L,), f32)
        return 0

    lax.fori_loop(0, NP // L, z, 0)
    pltpu.sync_copy(edges_hbm.at[pl.ds(c * 1280 + s * 80, 80)], edgebuf)
    ones = jnp.full((L,), 1.0, f32)

    def body(r, _):
        for q in range(8):
            v = edgebuf[r, pl.ds(q * L, L)]
            plsc.addupdate_scatter(cnt, [v], ones)
        return 0

    lax.fori_loop(0, 80, body, 0)
    pltpu.sync_copy(cnt, out_hbm.at[pl.ds((c * NS + s) * NP, NP)])


@jax.jit
def _sc_degrees(edges2d):
    return pl.kernel(
        _deg_body,
        out_type=jax.ShapeDtypeStruct((32 * NP,), f32),
        mesh=_mesh,
        scratch_types=[
            pltpu.VMEM((80, 128), i32),
            pltpu.VMEM((NP,), f32),
        ],
        compiler_params=_params,
    )(edges2d)


def _hist_body(nft_ref, deg_ref, h_ref, norms_ref):
    degp = deg_ref[...]
    d_src = jnp.sum(degp[:NS], axis=0)
    d_dst = jnp.sum(degp[NS:], axis=0)
    ns = jnp.where(d_src > 0, lax.rsqrt(jnp.maximum(d_src, 1.0)), 0.0)
    nd = jnp.where(d_dst > 0, lax.rsqrt(jnp.maximum(d_dst, 1.0)), 0.0)
    norms_ref[...] = jnp.stack([ns, nd], axis=0)
    feat = nft_ref[...]
    io = lax.broadcasted_iota(i32, (F, feat.shape[1]), 0)
    acc = jnp.zeros((F, feat.shape[1]), f32)
    for j in range(10):
        acc = acc + (feat[j : j + 1, :] == io).astype(f32)
    h_ref[...] = acc * ns[None, :]


@jax.jit
def _tc_hist(nft, deg):
    blk = 2048
    return pl.pallas_call(
        _hist_body,
        grid=(NP // blk,),
        in_specs=[
            pl.BlockSpec((10, blk), lambda j: (0, j)),
            pl.BlockSpec((32, blk), lambda j: (0, j)),
        ],
        out_specs=[
            pl.BlockSpec((F, blk), lambda j: (0, j)),
            pl.BlockSpec((2, blk), lambda j: (0, j)),
        ],
        out_shape=[
            jax.ShapeDtypeStruct((F, NP), f32),
            jax.ShapeDtypeStruct((2, NP), f32),
        ],
    )(nft, deg)


def _accum_edges(src2d, dst2d, hsl, acc, srcb, dstb, sem0, sem1):
    pdiv, pmod = _pats4()
    pnp = pmod * SAG
    sems = (sem0, sem1)

    def start(b, slot):
        pltpu.async_copy(src2d.at[pl.ds(b * 40, 40)], srcb.at[slot],
                         sems[slot])
        pltpu.async_copy(dst2d.at[pl.ds(b * 40, 40)], dstb.at[slot],
                         sems[slot])

    def drain(slot):
        pltpu.make_async_copy(src2d.at[pl.ds(0, 40)], srcb.at[slot],
                              sems[slot]).wait()
        pltpu.make_async_copy(dst2d.at[pl.ds(0, 40)], dstb.at[slot],
                              sems[slot]).wait()

    def process(slot):
        slotv = jnp.full((L,), slot, i32)

        def row(g, _):
            rowv = jnp.zeros((L,), i32) + g
            for e in range(32):
                colv = pdiv + 4 * e
                se = plsc.load_gather(srcb, [slotv, rowv, colv])
                de = plsc.load_gather(dstb, [slotv, rowv, colv])
                val = plsc.load_gather(hsl, [se + pnp])
                plsc.addupdate_scatter(acc, [de + pnp], val)
            return 0

        lax.fori_loop(0, 40, row, 0)

    start(0, 0)

    def blk(i, _):
        start(2 * i + 1, 1)
        drain(0)
        process(0)

        @pl.when(i < 15)
        def _():
            start(2 * i + 2, 0)

        drain(1)
        process(1)
        return 0

    lax.fori_loop(0, 16, blk, 0)


SLW = 4 * NP
SAG = NP + 8
SBUFW = 4 * SAG


def _ldsl(hbm, buf, q):
    for k in range(4):
        pltpu.sync_copy(hbm.at[pl.ds((q * 4 + k) * NP, NP)],
                        buf.at[pl.ds(k * SAG, NP)])


def _stsl(buf, hbm, q):
    for k in range(4):
        pltpu.sync_copy(buf.at[pl.ds(k * SAG, NP)],
                        hbm.at[pl.ds((q * 4 + k) * NP, NP)])


def _agg1_body(hp, src2d, dst2d, out_hbm, hsl, acc, srcb, dstb, sem0, sem1):
    c = lax.axis_index("c")
    s = lax.axis_index("s")
    q = c * NS + s
    _ldsl(hp, hsl, q)
    _zero_flat(acc, SBUFW)
    _accum_edges(src2d, dst2d, hsl, acc, srcb, dstb, sem0, sem1)
    _stsl(acc, out_hbm, q)


@jax.jit
def _sc_agg1(hp, src2d, dst2d):
    return pl.kernel(
        _agg1_body,
        out_type=jax.ShapeDtypeStruct((32 * SLW,), f32),
        mesh=_mesh,
        scratch_types=[
            pltpu.VMEM((SBUFW,), f32),
            pltpu.VMEM((SBUFW,), f32),
            pltpu.VMEM((2, 40, 128), i32),
            pltpu.VMEM((2, 40, 128), i32),
            pltpu.SemaphoreType.DMA,
            pltpu.SemaphoreType.DMA,
        ],
        compiler_params=_params,
    )(hp, src2d, dst2d)


def _agg2_body(hp, src2d, dst2d, out_hbm, hsl, acc, srcb, dstb, sem0, sem1):
    c = lax.axis_index("c")
    s = lax.axis_index("s")
    q = c * NS + s
    for r in range(2):
        q2 = q + 32 * r
        _ldsl(hp, hsl, q2)
        _zero_flat(acc, SBUFW)
        _accum_edges(src2d, dst2d, hsl, acc, srcb, dstb, sem0, sem1)
        _stsl(acc, out_hbm, q2)


@jax.jit
def _sc_agg2(hp, src2d, dst2d):
    return pl.kernel(
        _agg2_body,
        out_type=jax.ShapeDtypeStruct((64 * SLW,), f32),
        mesh=_mesh,
        scratch_types=[
            pltpu.VMEM((SBUFW,), f32),
            pltpu.VMEM((SBUFW,), f32),
            pltpu.VMEM((2, 40, 128), i32),
            pltpu.VMEM((2, 40, 128), i32),
            pltpu.SemaphoreType.DMA,
            pltpu.SemaphoreType.DMA,
        ],
        compiler_params=_params,
    )(hp, src2d, dst2d)


def _l1_body(p_ref, norms_ref, w_ref, b_ref, out_ref):
    nm = norms_ref[...]
    agg = p_ref[...] * nm[1][None, :]
    x = lax.dot_general(
        w_ref[...], agg, (((0,), (0,)), ((), ())),
        preferred_element_type=f32,
    ) + b_ref[...].reshape(D, 1)
    out_ref[...] = jnp.maximum(x, 0.0) * nm[0][None, :]


@jax.jit
def _tc_l1(P, norms, W1, b1):
    blk = 1024
    return pl.pallas_call(
        _l1_body,
        grid=(NP // blk,),
        in_specs=[
            pl.BlockSpec((F, blk), lambda j: (0, j)),
            pl.BlockSpec((2, blk), lambda j: (0, j)),
            pl.BlockSpec((F, D), lambda j: (0, 0)),
            pl.BlockSpec((1, D), lambda j: (0, 0)),
        ],
        out_specs=pl.BlockSpec((D, blk), lambda j: (0, j)),
        out_shape=jax.ShapeDtypeStruct((D, NP), f32),
    )(P, norms, W1, b1)


def _l2_body(a_ref, norms_ref, w_ref, b_ref, h2_ref, s_ref):
    j = pl.program_id(0)
    nm = norms_ref[...]
    h2 = lax.dot_general(
        w_ref[...], a_ref[...] * nm[1][None, :], (((0,), (0,)), ((), ())),
        preferred_element_type=f32,
    ) + b_ref[...].reshape(D, 1)
    h2_ref[...] = h2
    rn = jnp.sqrt(jnp.sum(h2 * h2, axis=0))
    cols = j * h2.shape[1] + lax.broadcasted_iota(i32, rn.shape, 0)
    contrib = jnp.sum(jnp.where(cols < N, rn, 0.0))

    @pl.when(j == 0)
    def _():
        s_ref[...] = jnp.zeros((1, 1), f32)

    s_ref[...] += jnp.full((1, 1), 1.0, f32) * contrib


@jax.jit
def _tc_l2(A2, norms, W2, b2):
    blk = 1024
    return pl.pallas_call(
        _l2_body,
        grid=(NP // blk,),
        in_specs=[
            pl.BlockSpec((D, blk), lambda j: (0, j)),
            pl.BlockSpec((2, blk), lambda j: (0, j)),
            pl.BlockSpec((D, D), lambda j: (0, 0)),
            pl.BlockSpec((1, D), lambda j: (0, 0)),
        ],
        out_specs=[
            pl.BlockSpec((D, blk), lambda j: (0, j)),
            pl.BlockSpec((1, 1), lambda j: (0, 0)),
        ],
        out_shape=[
            jax.ShapeDtypeStruct((D, NP), f32),
            jax.ShapeDtypeStruct((1, 1), f32),
        ],
    )(A2, norms, W2, b2)


def _segsum_body(h2_hbm, gid_hbm, out_hbm, sbuf, gidb, acc):
    c = lax.axis_index("c")
    s = lax.axis_index("s")
    q = c * NS + s
    _zero_flat(acc, 8 * NMOLP)
    for k in range(8):
        pltpu.sync_copy(h2_hbm.at[pl.ds((q * 8 + k) * NP, NP)],
                        sbuf.at[pl.ds(k * SAG, NP)])
    pltpu.sync_copy(gid_hbm, gidb)
    pat2, pat8 = _pats()
    base = pat8 * SAG + pat2

    def body(jj, _):
        rv = pat2 + 2 * jj
        gv = plsc.load_gather(gidb, [rv])
        val = plsc.load_gather(sbuf, [base + 2 * jj])
        plsc.addupdate_scatter(acc, [gv * 8 + pat8], val)
        return 0

    lax.fori_loop(0, NP // 2, body, 0)
    pltpu.sync_copy(acc, out_hbm.at[pl.ds(q * 8 * NMOLP, 8 * NMOLP)])


@jax.jit
def _sc_segsum(h2t, gid1d):
    return pl.kernel(
        _segsum_body,
        out_type=jax.ShapeDtypeStruct((32 * 8 * NMOLP,), f32),
        mesh=_mesh,
        scratch_types=[
            pltpu.VMEM((8 * SAG,), f32),
            pltpu.VMEM((NP,), i32),
            pltpu.VMEM((8 * NMOLP,), f32),
        ],
        compiler_params=_params,
    )(h2t, gid1d)


def _combine_body(p_ref, s_ref, out_ref):
    factor = (16.0 * N) / s_ref[0, 0]
    out_ref[...] = p_ref[...] * factor


@jax.jit
def _tc_combine(embP, S):
    blk = 256
    return pl.pallas_call(
        _combine_body,
        grid=(NMOLP // blk,),
        in_specs=[
            pl.BlockSpec((blk, D), lambda j: (j, 0)),
            pl.BlockSpec((1, 1), lambda j: (0, 0)),
        ],
        out_specs=pl.BlockSpec((blk, D), lambda j: (j, 0)),
        out_shape=jax.ShapeDtypeStruct((NMOLP, D), f32),
    )(embP, S)


def _moltable_body(emb_hbm, adj_hbm, out_hbm, adjb, selfb, neib, outb, sem):
    c = lax.axis_index("c")
    s = lax.axis_index("s")
    w = c * NS + s
    pltpu.sync_copy(adj_hbm.at[pl.ds(w * 512, 512)], adjb)
    pltpu.sync_copy(emb_hbm.at[pl.ds(w * 64, 64)], selfb)
    for q in range(4):
        pltpu.async_copy(
            emb_hbm.at[adjb.at[pl.ds(q * 128, 128)]], neib, sem
        ).wait()

        def item(b16, _):
            b = q * 16 + b16
            for v in range(16):
                sl = pl.ds(v * L, L)
                acc = neib[b16 * 8, sl]
                for jj in range(1, 8):
                    acc = acc + neib[b16 * 8 + jj, sl]
                outb[b, sl] = selfb[b, sl] * 2.0 + acc * 0.125
            return 0

        lax.fori_loop(0, 16, item, 0)
    pltpu.sync_copy(outb, out_hbm.at[pl.ds(w * 64, 64)])


@jax.jit
def _sc_moltable(emb, flatadj):
    return pl.kernel(
        _moltable_body,
        out_type=jax.ShapeDtypeStruct((NMOLP, D), f32),
        mesh=_mesh,
        scratch_types=[
            pltpu.VMEM((512,), i32),
            pltpu.VMEM((64, D), f32),
            pltpu.VMEM((128, D), f32),
            pltpu.VMEM((64, D), f32),
            pltpu.SemaphoreType.DMA,
        ],
        compiler_params=_params,
    )(emb, flatadj)


def _bgather_body(g_hbm, ids_hbm, out_hbm, idsb, gbuf, sem):
    c = lax.axis_index("c")
    s = lax.axis_index("s")
    w = c * NS + s
    pltpu.sync_copy(ids_hbm.at[pl.ds(w * 256, 256)], idsb)
    for q in range(2):
        pltpu.async_copy(
            g_hbm.at[idsb.at[pl.ds(q * 128, 128)]], gbuf, sem
        ).wait()
        pltpu.sync_copy(gbuf, out_hbm.at[pl.ds(w * 256 + q * 128, 128)])


@jax.jit
def _sc_bgather(G, ids):
    return pl.kernel(
        _bgather_body,
        out_type=jax.ShapeDtypeStruct((2 * BATCH, D), f32),
        mesh=_mesh,
        scratch_types=[
            pltpu.VMEM((256,), i32),
            pltpu.VMEM((128, D), f32),
            pltpu.SemaphoreType.DMA,
        ],
        compiler_params=_params,
    )(G, ids)


def kernel(r_id, p_id, r_bond, p_bond, node_feature, edge_index, graph_id,
           adj_graph, bond_feature, W1, b1, W2, b2, rel_para):
    src = edge_index[0].astype(i32)
    dst = edge_index[1].astype(i32)
    pad = jnp.full((NEP - NE,), PADN, i32)
    src2d = jnp.concatenate([src, pad]).reshape(NEP // 128, 128)
    dst2d = jnp.concatenate([dst, pad]).reshape(NEP // 128, 128)
    edges2d = jnp.concatenate([src2d, dst2d], axis=0)
    nf_p = jnp.pad(node_feature.astype(i32), ((0, NP - N), (0, 0)))
    gid1d = jnp.concatenate(
        [graph_id.astype(i32), jnp.full((NP - N,), PADG, i32)]
    )
    flatadj = jnp.pad(
        adj_graph.astype(i32), ((0, NMOLP - NMOL), (0, 0))
    ).reshape(-1)
    ids = jnp.concatenate([r_id, p_id]).astype(i32)

    deg = _sc_degrees(edges2d)
    hT, norms = _tc_hist(nf_p.T, deg.reshape(32, NP))
    P = _sc_agg1(hT.reshape(-1), src2d, dst2d)
    h1T = _tc_l1(P.reshape(F, NP), norms, W1, b1.reshape(1, D))
    A2 = _sc_agg2(h1T.reshape(-1), src2d, dst2d)
    h2T, S = _tc_l2(A2.reshape(D, NP), norms, W2, b2.reshape(1, D))
    embP = _sc_segsum(h2T.reshape(-1), gid1d)
    emb = embP.reshape(32, NMOLP, 8).transpose(1, 0, 2).reshape(NMOLP, D)
    emb = _tc_combine(emb, S)
    G = _sc_moltable(emb, flatadj)
    out = _sc_bgather(G, ids)
    return out[:BATCH], out[BATCH:]


# speedup vs baseline: 5.0919x; 1.3865x over previous
import jax
import jax.numpy as jnp
from jax import lax
from jax.experimental import pallas as pl
from jax.experimental.pallas import tpu as pltpu
from jax.experimental.pallas import tpu_sc as plsc

N = 10000
NP = 10240
NPH = NP // 2
NE = 160000
NEP = 163840
F = 128
D = 256
NMOL = 2000
NMOLP = 2048
BATCH = 4096
PADN = NP - 1
PADG = NMOLP - 1

NC = 2
NS = 16
L = 16

_mesh = plsc.VectorSubcoreMesh(core_axis_name="c", subcore_axis_name="s")
_params = pltpu.CompilerParams(needs_layout_passes=False)
f32 = jnp.float32
i32 = jnp.int32


def _pats():
    io = lax.iota(i32, L)
    pat2 = lax.shift_right_logical(io, 3)
    pat8 = lax.bitwise_and(io, 7)
    return pat2, pat8


def _pats4():
    io = lax.iota(i32, L)
    pdiv = lax.shift_right_logical(io, 2)
    pmod = lax.bitwise_and(io, 3)
    return pdiv, pmod


def _zero_flat(acc, n):
    z = jnp.zeros((L,), f32)

    def body(i, _):
        acc[pl.ds(i * L, L)] = z
        return 0

    lax.fori_loop(0, n // L, body, 0)


def _deg_body(edges_hbm, out_hbm, edgebuf, cnt):
    c = lax.axis_index("c")
    s = lax.axis_index("s")

    def z(i, _):
        cnt[pl.ds(i * L, L)] = jnp.zeros((L,), f32)
        return 0

    lax.fori_loop(0, NP // L, z, 0)
    pltpu.sync_copy(edges_hbm.at[pl.ds(c * 1280 + s * 80, 80)], edgebuf)
    ones = jnp.full((L,), 1.0, f32)

    def body(r, _):
        for q in range(8):
            v = edgebuf[r, pl.ds(q * L, L)]
            plsc.addupdate_scatter(cnt, [v], ones)
        return 0

    lax.fori_loop(0, 80, body, 0)
    pltpu.sync_copy(cnt, out_hbm.at[pl.ds((c * NS + s) * NP, NP)])


@jax.jit
def _sc_degrees(edges2d):
    return pl.kernel(
        _deg_body,
        out_type=jax.ShapeDtypeStruct((32 * NP,), f32),
        mesh=_mesh,
        scratch_types=[
            pltpu.VMEM((80, 128), i32),
            pltpu.VMEM((NP,), f32),
        ],
        compiler_params=_params,
    )(edges2d)


def _hist_body(nft_ref, deg_ref, h_ref, norms_ref):
    degp = deg_ref[...]
    d_src = jnp.sum(degp[:NS], axis=0)
    d_dst = jnp.sum(degp[NS:], axis=0)
    ns = jnp.where(d_src > 0, lax.rsqrt(jnp.maximum(d_src, 1.0)), 0.0)
    nd = jnp.where(d_dst > 0, lax.rsqrt(jnp.maximum(d_dst, 1.0)), 0.0)
    norms_ref[...] = jnp.stack([ns, nd], axis=0)
    feat = nft_ref[...]
    io = lax.broadcasted_iota(i32, (F, feat.shape[1]), 0)
    acc = jnp.zeros((F, feat.shape[1]), f32)
    for j in range(10):
        acc = acc + (feat[j : j + 1, :] == io).astype(f32)
    h_ref[...] = acc * ns[None, :]


@jax.jit
def _tc_hist(nft, deg):
    blk = 2048
    return pl.pallas_call(
        _hist_body,
        grid=(NP // blk,),
        in_specs=[
            pl.BlockSpec((10, blk), lambda j: (0, j)),
            pl.BlockSpec((32, blk), lambda j: (0, j)),
        ],
        out_specs=[
            pl.BlockSpec((F, blk), lambda j: (0, j)),
            pl.BlockSpec((2, blk), lambda j: (0, j)),
        ],
        out_shape=[
            jax.ShapeDtypeStruct((F, NP), f32),
            jax.ShapeDtypeStruct((2, NP), f32),
        ],
    )(nft, deg)


def _accum_edges(src2d, dst2d, hsl, acc, srcb, dstb, sem0, sem1):
    sems = (sem0, sem1)

    def start(b, slot):
        pltpu.async_copy(src2d.at[pl.ds(b * 40, 40)], srcb.at[slot],
                         sems[slot])
        pltpu.async_copy(dst2d.at[pl.ds(b * 40, 40)], dstb.at[slot],
                         sems[slot])

    def drain(slot):
        pltpu.make_async_copy(src2d.at[pl.ds(0, 40)], srcb.at[slot],
                              sems[slot]).wait()
        pltpu.make_async_copy(dst2d.at[pl.ds(0, 40)], dstb.at[slot],
                              sems[slot]).wait()

    def process(slot):
        def row(g, _):
            for i in range(8):
                se = srcb[slot, g, pl.ds(i * L, L)]
                de = dstb[slot, g, pl.ds(i * L, L)]
                for k in range(4):
                    si = se if k == 0 else se + (k * SAG)
                    di = de if k == 0 else de + (k * SAG)
                    val = plsc.load_gather(hsl, [si])
                    plsc.addupdate_scatter(acc, [di], val)
            return 0

        lax.fori_loop(0, 40, row, 0)

    start(0, 0)

    def blk(i, _):
        start(2 * i + 1, 1)
        drain(0)
        process(0)

        @pl.when(i < 15)
        def _():
            start(2 * i + 2, 0)

        drain(1)
        process(1)
        return 0

    lax.fori_loop(0, 16, blk, 0)


SLW = 4 * NP
SAG = NP + 8
SBUFW = 4 * SAG


def _ldsl(hbm, buf, q):
    for k in range(4):
        pltpu.sync_copy(hbm.at[pl.ds((q * 4 + k) * NP, NP)],
                        buf.at[pl.ds(k * SAG, NP)])


def _stsl(buf, hbm, q):
    for k in range(4):
        pltpu.sync_copy(buf.at[pl.ds(k * SAG, NP)],
                        hbm.at[pl.ds((q * 4 + k) * NP, NP)])


def _agg1_body(hp, src2d, dst2d, out_hbm, hsl, acc, srcb, dstb, sem0, sem1):
    c = lax.axis_index("c")
    s = lax.axis_index("s")
    q = c * NS + s
    _ldsl(hp, hsl, q)
    _zero_flat(acc, SBUFW)
    _accum_edges(src2d, dst2d, hsl, acc, srcb, dstb, sem0, sem1)
    _stsl(acc, out_hbm, q)


@jax.jit
def _sc_agg1(hp, src2d, dst2d):
    return pl.kernel(
        _agg1_body,
        out_type=jax.ShapeDtypeStruct((32 * SLW,), f32),
        mesh=_mesh,
        scratch_types=[
            pltpu.VMEM((SBUFW,), f32),
            pltpu.VMEM((SBUFW,), f32),
            pltpu.VMEM((2, 40, 128), i32),
            pltpu.VMEM((2, 40, 128), i32),
            pltpu.SemaphoreType.DMA,
            pltpu.SemaphoreType.DMA,
        ],
        compiler_params=_params,
    )(hp, src2d, dst2d)


def _agg2_body(hp, src2d, dst2d, out_hbm, hsl, acc, srcb, dstb, sem0, sem1):
    c = lax.axis_index("c")
    s = lax.axis_index("s")
    q = c * NS + s
    for r in range(2):
        q2 = q + 32 * r
        _ldsl(hp, hsl, q2)
        _zero_flat(acc, SBUFW)
        _accum_edges(src2d, dst2d, hsl, acc, srcb, dstb, sem0, sem1)
        _stsl(acc, out_hbm, q2)


@jax.jit
def _sc_agg2(hp, src2d, dst2d):
    return pl.kernel(
        _agg2_body,
        out_type=jax.ShapeDtypeStruct((64 * SLW,), f32),
        mesh=_mesh,
        scratch_types=[
            pltpu.VMEM((SBUFW,), f32),
            pltpu.VMEM((SBUFW,), f32),
            pltpu.VMEM((2, 40, 128), i32),
            pltpu.VMEM((2, 40, 128), i32),
            pltpu.SemaphoreType.DMA,
            pltpu.SemaphoreType.DMA,
        ],
        compiler_params=_params,
    )(hp, src2d, dst2d)


def _l1_body(p_ref, norms_ref, w_ref, b_ref, out_ref):
    nm = norms_ref[...]
    agg = p_ref[...] * nm[1][None, :]
    x = lax.dot_general(
        w_ref[...], agg, (((0,), (0,)), ((), ())),
        preferred_element_type=f32,
    ) + b_ref[...].reshape(D, 1)
    out_ref[...] = jnp.maximum(x, 0.0) * nm[0][None, :]


@jax.jit
def _tc_l1(P, norms, W1, b1):
    blk = 1024
    return pl.pallas_call(
        _l1_body,
        grid=(NP // blk,),
        in_specs=[
            pl.BlockSpec((F, blk), lambda j: (0, j)),
            pl.BlockSpec((2, blk), lambda j: (0, j)),
            pl.BlockSpec((F, D), lambda j: (0, 0)),
            pl.BlockSpec((1, D), lambda j: (0, 0)),
        ],
        out_specs=pl.BlockSpec((D, blk), lambda j: (0, j)),
        out_shape=jax.ShapeDtypeStruct((D, NP), f32),
    )(P, norms, W1, b1)


def _l2_body(a_ref, norms_ref, w_ref, b_ref, h2_ref, s_ref):
    j = pl.program_id(0)
    nm = norms_ref[...]
    h2 = lax.dot_general(
        w_ref[...], a_ref[...] * nm[1][None, :], (((0,), (0,)), ((), ())),
        preferred_element_type=f32,
    ) + b_ref[...].reshape(D, 1)
    h2_ref[...] = h2
    rn = jnp.sqrt(jnp.sum(h2 * h2, axis=0))
    cols = j * h2.shape[1] + lax.broadcasted_iota(i32, rn.shape, 0)
    contrib = jnp.sum(jnp.where(cols < N, rn, 0.0))

    @pl.when(j == 0)
    def _():
        s_ref[...] = jnp.zeros((1, 1), f32)

    s_ref[...] += jnp.full((1, 1), 1.0, f32) * contrib


@jax.jit
def _tc_l2(A2, norms, W2, b2):
    blk = 1024
    return pl.pallas_call(
        _l2_body,
        grid=(NP // blk,),
        in_specs=[
            pl.BlockSpec((D, blk), lambda j: (0, j)),
            pl.BlockSpec((2, blk), lambda j: (0, j)),
            pl.BlockSpec((D, D), lambda j: (0, 0)),
            pl.BlockSpec((1, D), lambda j: (0, 0)),
        ],
        out_specs=[
            pl.BlockSpec((D, blk), lambda j: (0, j)),
            pl.BlockSpec((1, 1), lambda j: (0, 0)),
        ],
        out_shape=[
            jax.ShapeDtypeStruct((D, NP), f32),
            jax.ShapeDtypeStruct((1, 1), f32),
        ],
    )(A2, norms, W2, b2)


def _segsum_body(h2_hbm, gid_hbm, out_hbm, sbuf, gidb, acc):
    c = lax.axis_index("c")
    s = lax.axis_index("s")
    q = c * NS + s
    _zero_flat(acc, 8 * NMOLP)
    for k in range(8):
        pltpu.sync_copy(h2_hbm.at[pl.ds((q * 8 + k) * NP, NP)],
                        sbuf.at[pl.ds(k * SAG, NP)])
    pltpu.sync_copy(gid_hbm, gidb)
    pat2, pat8 = _pats()
    base = pat8 * SAG + pat2

    def body(jj, _):
        rv = pat2 + 2 * jj
        gv = plsc.load_gather(gidb, [rv])
        val = plsc.load_gather(sbuf, [base + 2 * jj])
        plsc.addupdate_scatter(acc, [gv * 8 + pat8], val)
        return 0

    lax.fori_loop(0, NP // 2, body, 0)
    pltpu.sync_copy(acc, out_hbm.at[pl.ds(q * 8 * NMOLP, 8 * NMOLP)])


@jax.jit
def _sc_segsum(h2t, gid1d):
    return pl.kernel(
        _segsum_body,
        out_type=jax.ShapeDtypeStruct((32 * 8 * NMOLP,), f32),
        mesh=_mesh,
        scratch_types=[
            pltpu.VMEM((8 * SAG,), f32),
            pltpu.VMEM((NP,), i32),
            pltpu.VMEM((8 * NMOLP,), f32),
        ],
        compiler_params=_params,
    )(h2t, gid1d)


def _combine_body(p_ref, s_ref, out_ref):
    factor = (16.0 * N) / s_ref[0, 0]
    out_ref[...] = p_ref[...] * factor


@jax.jit
def _tc_combine(embP, S):
    blk = 256
    return pl.pallas_call(
        _combine_body,
        grid=(NMOLP // blk,),
        in_specs=[
            pl.BlockSpec((blk, D), lambda j: (j, 0)),
            pl.BlockSpec((1, 1), lambda j: (0, 0)),
        ],
        out_specs=pl.BlockSpec((blk, D), lambda j: (j, 0)),
        out_shape=jax.ShapeDtypeStruct((NMOLP, D), f32),
    )(embP, S)


def _moltable_body(emb_hbm, adj_hbm, out_hbm, adjb, selfb, neib, outb, sem):
    c = lax.axis_index("c")
    s = lax.axis_index("s")
    w = c * NS + s
    pltpu.sync_copy(adj_hbm.at[pl.ds(w * 512, 512)], adjb)
    pltpu.sync_copy(emb_hbm.at[pl.ds(w * 64, 64)], selfb)
    for q in range(4):
        pltpu.async_copy(
            emb_hbm.at[adjb.at[pl.ds(q * 128, 128)]], neib, sem
        ).wait()

        def item(b16, _):
            b = q * 16 + b16
            for v in range(16):
                sl = pl.ds(v * L, L)
                acc = neib[b16 * 8, sl]
                for jj in range(1, 8):
                    acc = acc + neib[b16 * 8 + jj, sl]
                outb[b, sl] = selfb[b, sl] * 2.0 + acc * 0.125
            return 0

        lax.fori_loop(0, 16, item, 0)
    pltpu.sync_copy(outb, out_hbm.at[pl.ds(w * 64, 64)])


@jax.jit
def _sc_moltable(emb, flatadj):
    return pl.kernel(
        _moltable_body,
        out_type=jax.ShapeDtypeStruct((NMOLP, D), f32),
        mesh=_mesh,
        scratch_types=[
            pltpu.VMEM((512,), i32),
            pltpu.VMEM((64, D), f32),
            pltpu.VMEM((128, D), f32),
            pltpu.VMEM((64, D), f32),
            pltpu.SemaphoreType.DMA,
        ],
        compiler_params=_params,
    )(emb, flatadj)


def _bgather_body(g_hbm, ids_hbm, out_hbm, idsb, gbuf, sem):
    c = lax.axis_index("c")
    s = lax.axis_index("s")
    w = c * NS + s
    pltpu.sync_copy(ids_hbm.at[pl.ds(w * 256, 256)], idsb)
    for q in range(2):
        pltpu.async_copy(
            g_hbm.at[idsb.at[pl.ds(q * 128, 128)]], gbuf, sem
        ).wait()
        pltpu.sync_copy(gbuf, out_hbm.at[pl.ds(w * 256 + q * 128, 128)])


@jax.jit
def _sc_bgather(G, ids):
    return pl.kernel(
        _bgather_body,
        out_type=jax.ShapeDtypeStruct((2 * BATCH, D), f32),
        mesh=_mesh,
        scratch_types=[
            pltpu.VMEM((256,), i32),
            pltpu.VMEM((128, D), f32),
            pltpu.SemaphoreType.DMA,
        ],
        compiler_params=_params,
    )(G, ids)


def kernel(r_id, p_id, r_bond, p_bond, node_feature, edge_index, graph_id,
           adj_graph, bond_feature, W1, b1, W2, b2, rel_para):
    src = edge_index[0].astype(i32)
    dst = edge_index[1].astype(i32)
    pad = jnp.full((NEP - NE,), PADN, i32)
    src2d = jnp.concatenate([src, pad]).reshape(NEP // 128, 128)
    dst2d = jnp.concatenate([dst, pad]).reshape(NEP // 128, 128)
    edges2d = jnp.concatenate([src2d, dst2d], axis=0)
    nf_p = jnp.pad(node_feature.astype(i32), ((0, NP - N), (0, 0)))
    gid1d = jnp.concatenate(
        [graph_id.astype(i32), jnp.full((NP - N,), PADG, i32)]
    )
    flatadj = jnp.pad(
        adj_graph.astype(i32), ((0, NMOLP - NMOL), (0, 0))
    ).reshape(-1)
    ids = jnp.concatenate([r_id, p_id]).astype(i32)

    deg = _sc_degrees(edges2d)
    hT, norms = _tc_hist(nf_p.T, deg.reshape(32, NP))
    P = _sc_agg1(hT.reshape(-1), src2d, dst2d)
    h1T = _tc_l1(P.reshape(F, NP), norms, W1, b1.reshape(1, D))
    A2 = _sc_agg2(h1T.reshape(-1), src2d, dst2d)
    h2T, S = _tc_l2(A2.reshape(D, NP), norms, W2, b2.reshape(1, D))
    embP = _sc_segsum(h2T.reshape(-1), gid1d)
    emb = embP.reshape(32, NMOLP, 8).transpose(1, 0, 2).reshape(NMOLP, D)
    emb = _tc_combine(emb, S)
    G = _sc_moltable(emb, flatadj)
    out = _sc_bgather(G, ids)
    return out[:BATCH], out[BATCH:]
